# baseline (speedup 1.0000x reference)
import functools

import jax
import jax.numpy as jnp
from jax import lax
from jax.experimental import pallas as pl
from jax.experimental.pallas import tpu as pltpu
from jax.experimental.pallas import tpu_sc as plsc

NUM_EMB = 1000000
HALF_EMB = NUM_EMB // 2
SUB = 32
BATCH = 16384
NC, NS = 2, 16
NW = NC * NS
BPW = BATCH // NW
CHUNK = 128
NCH = BPW // CHUNK
HALF = BPW // 2
IGN = 2**30

_mesh = plsc.VectorSubcoreMesh(core_axis_name="c", subcore_axis_name="s")


@functools.partial(
    pl.kernel,
    mesh=_mesh,
    compiler_params=pltpu.CompilerParams(use_tc_tiling_on_sc=False),
    out_type=jax.ShapeDtypeStruct((BATCH, 2 * SUB), jnp.float32),
    scratch_types=[
        pltpu.VMEM((NCH, CHUNK), jnp.int32),
        pltpu.VMEM((NCH, CHUNK), jnp.int32),
        pltpu.VMEM((HALF, SUB), jnp.float32),
        pltpu.VMEM((HALF, SUB), jnp.float32),
        pltpu.VMEM((HALF, 2 * SUB), jnp.float32),
        pltpu.SemaphoreType.DMA,
    ],
)
def _hash_embed(idx0_hbm, idx1_hbm, t0a_hbm, t0b_hbm, t1a_hbm, t1b_hbm,
                out_hbm, idx0_v, idx1_v, rows0_v, rows1_v, out_v, sem):
    wid = lax.axis_index("s") * NC + lax.axis_index("c")
    base = wid * BPW
    pltpu.sync_copy(idx0_hbm.at[wid], idx0_v)
    pltpu.sync_copy(idx1_hbm.at[wid], idx1_v)
    for half in range(2):
        copies = []
        for jj in range(HALF // 16):
            j = half * (HALF // CHUNK) + jj // (CHUNK // 16)
            k = jj % (CHUNK // 16)
            dst0 = rows0_v.at[pl.ds(jj * 16, 16)]
            dst1 = rows1_v.at[pl.ds(jj * 16, 16)]
            for (idx_v, ta, tb, dst) in (
                (idx0_v, t0a_hbm, t0b_hbm, dst0),
                (idx1_v, t1a_hbm, t1b_hbm, dst1),
            ):
                iv = idx_v[j, pl.ds(k * 16, 16)]
                iva = jnp.where(iv < HALF_EMB, iv, IGN)
                ivb = jnp.where(iv >= HALF_EMB, iv - HALF_EMB, IGN)
                copies.append(pltpu.async_copy(
                    ta.at[plsc.Indices(iva, ignored_value=IGN)], dst, sem))
                copies.append(pltpu.async_copy(
                    tb.at[plsc.Indices(ivb, ignored_value=IGN)], dst, sem))
        for c in copies:
            c.wait()

        @pl.loop(0, HALF)
        def _interleave(r):
            out_v[r, pl.ds(0, 16)] = rows0_v[r, pl.ds(0, 16)]
            out_v[r, pl.ds(16, 16)] = rows0_v[r, pl.ds(16, 16)]
            out_v[r, pl.ds(32, 16)] = rows1_v[r, pl.ds(0, 16)]
            out_v[r, pl.ds(48, 16)] = rows1_v[r, pl.ds(16, 16)]

        pltpu.sync_copy(out_v, out_hbm.at[pl.ds(base + half * HALF, HALF)])


def kernel(indices, table0, table1):
    idx = indices.astype(jnp.int32)
    idx0 = idx[0].reshape(NW, NCH, CHUNK)
    idx1 = idx[1].reshape(NW, NCH, CHUNK)
    return _hash_embed(idx0, idx1,
                       table0[:HALF_EMB], table0[HALF_EMB:],
                       table1[:HALF_EMB], table1[HALF_EMB:])


# speedup vs baseline: 1.5548x; 1.5548x over previous
import functools

import jax
import jax.numpy as jnp
from jax import lax
from jax.experimental import pallas as pl
from jax.experimental.pallas import tpu as pltpu
from jax.experimental.pallas import tpu_sc as plsc

NUM_EMB = 1000000
SUB = 32
BATCH = 16384
NC, NS = 2, 16
NW = NC * NS
BPW = BATCH // NW
HALF = BPW // 2
PACK = 128 // SUB
TROWS = NUM_EMB // PACK

_mesh = plsc.VectorSubcoreMesh(core_axis_name="c", subcore_axis_name="s")


@functools.partial(
    pl.kernel,
    mesh=_mesh,
    compiler_params=pltpu.CompilerParams(needs_layout_passes=False),
    out_type=jax.ShapeDtypeStruct((2 * SUB, BATCH), jnp.float32),
    scratch_types=[
        pltpu.VMEM((BPW // 128, 128), jnp.int32),
        pltpu.VMEM((BPW // 128, 128), jnp.int32),
        pltpu.VMEM((HALF, 128), jnp.float32),
        pltpu.VMEM((HALF, 128), jnp.float32),
        pltpu.VMEM((2 * SUB, BPW), jnp.float32),
        pltpu.SemaphoreType.DMA,
    ],
)
def _hash_embed(idx0_hbm, idx1_hbm, t0_hbm, t1_hbm, out_hbm,
                idx0_v, idx1_v, rows0_v, rows1_v, out_v, sem):
    wid = lax.axis_index("s") * NC + lax.axis_index("c")
    pltpu.sync_copy(idx0_hbm.at[wid], idx0_v)
    pltpu.sync_copy(idx1_hbm.at[wid], idx1_v)
    iota = lax.iota(jnp.int32, 16)

    for half in range(2):
        copies = []
        for g in range(HALF // 16):
            j = (half * HALF + g * 16) // 128
            o = (half * HALF + g * 16) % 128
            iv0 = lax.shift_right_logical(idx0_v[j, pl.ds(o, 16)], 2)
            iv1 = lax.shift_right_logical(idx1_v[j, pl.ds(o, 16)], 2)
            copies.append(pltpu.async_copy(
                t0_hbm.at[iv0], rows0_v.at[pl.ds(g * 16, 16)], sem))
            copies.append(pltpu.async_copy(
                t1_hbm.at[iv1], rows1_v.at[pl.ds(g * 16, 16)], sem))
        for c in copies:
            c.wait()
        for g in range(HALF // 16):
            j = (half * HALF + g * 16) // 128
            o = (half * HALF + g * 16) % 128
            rowids = g * 16 + iota
            colvec = half * HALF + g * 16 + iota
            for t, (idx_v, rows_v) in enumerate(
                ((idx0_v, rows0_v), (idx1_v, rows1_v))):
                lbase = lax.bitwise_and(idx_v[j, pl.ds(o, 16)], PACK - 1) * SUB
                for f in range(SUB):
                    vals = plsc.load_gather(rows_v, [rowids, lbase + f])
                    frow = lax.broadcast_in_dim(
                        jnp.int32(t * SUB + f), (16,), ())
                    plsc.store_scatter(out_v, [frow, colvec], vals)

    pltpu.sync_copy(out_v, out_hbm.at[:, pl.ds(wid * BPW, BPW)])


def kernel(indices, table0, table1):
    idx = indices.astype(jnp.int32)
    idx0 = idx[0].reshape(NW, BPW // 128, 128)
    idx1 = idx[1].reshape(NW, BPW // 128, 128)
    out_t = _hash_embed(idx0, idx1,
                        table0.reshape(TROWS, 128),
                        table1.reshape(TROWS, 128))
    return out_t.T
